# baseline (speedup 1.0000x reference)
import jax
import jax.numpy as jnp
from jax import lax
from jax.experimental import pallas as pl
from jax.experimental.pallas import tpu as pltpu
from jax.experimental.pallas import tpu_sc as plsc

B = 16384
D = 64
L = 16
NC = 2
NS = 16
NW = NC * NS
BPW = B // NW
CHUNK = 128
NCHUNK = BPW // CHUNK
GROUPS = BPW // L


def _body(uid_hbm, mid_hbm, ut_hbm, mt_hbm, out_hbm,
          uidx_v, midx_v, urows_v, mrows_v, res_v, sem):
    wid = lax.axis_index("s") * NC + lax.axis_index("c")
    base = wid * BPW

    for j in range(NCHUNK):
        pltpu.sync_copy(uid_hbm.at[pl.ds(base + j * CHUNK, CHUNK)], uidx_v.at[j])
        pltpu.sync_copy(mid_hbm.at[pl.ds(base + j * CHUNK, CHUNK)], midx_v.at[j])

    copies = []
    for j in range(NCHUNK):
        copies.append(pltpu.async_copy(
            ut_hbm.at[uidx_v.at[j]], urows_v.at[pl.ds(j * CHUNK, CHUNK)], sem))
        copies.append(pltpu.async_copy(
            mt_hbm.at[midx_v.at[j]], mrows_v.at[pl.ds(j * CHUNK, CHUNK)], sem))
    for c in copies:
        c.wait()

    def group(g, carry):
        rows = g * L + lax.iota(jnp.int32, L)

        def dstep(d, acc):
            cols = jnp.full((L,), d, dtype=jnp.int32)
            u = plsc.load_gather(urows_v, [rows, cols])
            m = plsc.load_gather(mrows_v, [rows, cols])
            return acc + u * m

        acc = lax.fori_loop(0, D, dstep, jnp.zeros((L,), jnp.float32))
        res_v[pl.ds(g * L, L)] = acc
        return carry

    lax.fori_loop(0, GROUPS, group, 0)

    pltpu.sync_copy(res_v, out_hbm.at[pl.ds(base, BPW)])


def kernel(user_ids, movie_ids, user_table, movie_table):
    mesh = plsc.VectorSubcoreMesh(core_axis_name="c", subcore_axis_name="s")
    k = pl.kernel(
        _body,
        out_type=jax.ShapeDtypeStruct((B,), jnp.float32),
        mesh=mesh,
        scratch_types=[
            pltpu.VMEM((NCHUNK, CHUNK), jnp.int32),
            pltpu.VMEM((NCHUNK, CHUNK), jnp.int32),
            pltpu.VMEM((BPW, D), jnp.float32),
            pltpu.VMEM((BPW, D), jnp.float32),
            pltpu.VMEM((BPW,), jnp.float32),
            pltpu.SemaphoreType.DMA,
        ],
        compiler_params=pltpu.CompilerParams(
            use_tc_tiling_on_sc=False, needs_layout_passes=False),
    )
    out = k(user_ids, movie_ids, user_table, movie_table)
    return out.reshape(B, 1)


# speedup vs baseline: 1.0082x; 1.0082x over previous
import jax
import jax.numpy as jnp
from jax import lax
from jax.experimental import pallas as pl
from jax.experimental.pallas import tpu as pltpu
from jax.experimental.pallas import tpu_sc as plsc

B = 16384
D = 64
L = 16
NC = 2
NS = 16
NW = NC * NS
BPW = B // NW
CHUNK = 128
NCHUNK = BPW // CHUNK
GPC = CHUNK // L


def _body(uid_hbm, mid_hbm, ut_hbm, mt_hbm, out_hbm,
          uidx_v, midx_v, urows_v, mrows_v, res_v, sem_i, *sems):
    wid = lax.axis_index("s") * NC + lax.axis_index("c")
    base = wid * BPW

    ci = pltpu.async_copy(uid_hbm.at[pl.ds(base, BPW)], uidx_v, sem_i)
    cm = pltpu.async_copy(mid_hbm.at[pl.ds(base, BPW)], midx_v, sem_i)
    ci.wait()
    cm.wait()

    copies = []
    for j in range(NCHUNK):
        copies.append(pltpu.async_copy(
            ut_hbm.at[uidx_v.at[pl.ds(j * CHUNK, CHUNK)]],
            urows_v.at[pl.ds(j * CHUNK, CHUNK)], sems[2 * j]))
        copies.append(pltpu.async_copy(
            mt_hbm.at[midx_v.at[pl.ds(j * CHUNK, CHUNK)]],
            mrows_v.at[pl.ds(j * CHUNK, CHUNK)], sems[2 * j + 1]))

    zero = jnp.zeros((L,), jnp.float32)
    for j in range(NCHUNK):
        copies[2 * j].wait()
        copies[2 * j + 1].wait()

        def group(g, carry, j=j):
            rows = j * CHUNK + g * L + lax.iota(jnp.int32, L)
            accs = [zero, zero, zero, zero]
            for d in range(D):
                cols = jnp.full((L,), d, dtype=jnp.int32)
                u = plsc.load_gather(urows_v, [rows, cols])
                m = plsc.load_gather(mrows_v, [rows, cols])
                accs[d % 4] = accs[d % 4] + u * m
            acc = (accs[0] + accs[1]) + (accs[2] + accs[3])
            res_v[pl.ds(j * CHUNK + g * L, L)] = acc
            return carry

        lax.fori_loop(0, GPC, group, 0)

    pltpu.sync_copy(res_v, out_hbm.at[pl.ds(base, BPW)])


def kernel(user_ids, movie_ids, user_table, movie_table):
    mesh = plsc.VectorSubcoreMesh(core_axis_name="c", subcore_axis_name="s")
    k = pl.kernel(
        _body,
        out_type=jax.ShapeDtypeStruct((B,), jnp.float32),
        mesh=mesh,
        scratch_types=[
            pltpu.VMEM((BPW,), jnp.int32),
            pltpu.VMEM((BPW,), jnp.int32),
            pltpu.VMEM((BPW, D), jnp.float32),
            pltpu.VMEM((BPW, D), jnp.float32),
            pltpu.VMEM((BPW,), jnp.float32),
            pltpu.SemaphoreType.DMA,
        ] + [pltpu.SemaphoreType.DMA] * (2 * NCHUNK),
        compiler_params=pltpu.CompilerParams(
            use_tc_tiling_on_sc=False, needs_layout_passes=False),
    )
    out = k(user_ids, movie_ids, user_table, movie_table)
    return out.reshape(B, 1)
